# baseline (speedup 1.0000x reference)
import functools

import jax
import jax.numpy as jnp
from jax import lax
from jax.experimental import pallas as pl
from jax.experimental.pallas import tpu as pltpu
from jax.experimental.pallas import tpu_sc as plsc

NC = 2
NS = 16
L = 16
CH = 128

_GATHER_DN = lax.GatherDimensionNumbers(
    offset_dims=(), collapsed_slice_dims=(0,), start_index_map=(0,))


def _perm(v, idx):
    return lax.gather(v, idx.reshape(L, 1), _GATHER_DN, slice_sizes=(1,),
                      mode=lax.GatherScatterMode.PROMISE_IN_BOUNDS)


def _transpose16(vs):
    lane = lax.iota(jnp.int32, L)
    for w in (1, 2, 4, 8):
        m = (lane & w) == 0
        new = list(vs)
        for i in range(L):
            if i & w == 0:
                j = i | w
                a, b = vs[i], vs[j]
                new[i] = jnp.where(m, a, _perm(b, lane ^ w))
                new[j] = jnp.where(m, _perm(a, lane ^ w), b)
        vs = new
    return vs


def _rsqrt_nr(m):
    i = plsc.bitcast(m, jnp.int32)
    seed = jnp.full((L,), 0x5F3759DF, jnp.int32) - lax.shift_right_logical(i, 1)
    y = plsc.bitcast(seed, jnp.float32)
    for _ in range(3):
        y = y * (1.5 - 0.5 * m * y * y)
    return y


@functools.lru_cache(maxsize=None)
def _build(B, D, V, margin):
    NW = NC * NS
    half = B // 2
    P = half // NW
    R = 2 * P
    NCH = R // CH
    G = P // L
    NCK = D // L

    mesh = plsc.VectorSubcoreMesh(
        core_axis_name="c", subcore_axis_name="s",
        num_cores=NC, num_subcores=NS)

    @functools.partial(
        pl.kernel,
        mesh=mesh,
        compiler_params=pltpu.CompilerParams(
            needs_layout_passes=False, use_tc_tiling_on_sc=False),
        out_type=(
            jax.ShapeDtypeStruct((half,), jnp.float32),
            jax.ShapeDtypeStruct((half,), jnp.float32),
            jax.ShapeDtypeStruct((NW, L), jnp.float32),
        ),
        scratch_types=[
            pltpu.VMEM((NCH, CH), jnp.int32),
            pltpu.VMEM((NCH, CH), jnp.int32),
            pltpu.VMEM((NCH, CH), jnp.int32),
            pltpu.VMEM((D * R,), jnp.float32),
            pltpu.VMEM((D * R,), jnp.float32),
            pltpu.VMEM((R, D), jnp.float32),
            pltpu.VMEM((R,), jnp.float32),
            pltpu.VMEM((L,), jnp.float32),
            pltpu.SemaphoreType.DMA,
        ],
    )
    def distmult(h_hbm, t_hbm, r_hbm, entT_hbm, rel_hbm,
                 pos_out, neg_out, loss_out,
                 idx_h, idx_t, idx_r, cols_h, cols_t, rows_r,
                 scores_v, lrow_v, sem):
        wid = lax.axis_index("s") * NC + lax.axis_index("c")
        pbase = wid * P
        nbase = half + wid * P

        for src, dst in ((h_hbm, idx_h), (t_hbm, idx_t), (r_hbm, idx_r)):
            for c in range(NCH // 2):
                pltpu.sync_copy(src.at[pl.ds(pbase + c * CH, CH)], dst.at[c])
            for c in range(NCH // 2):
                pltpu.sync_copy(src.at[pl.ds(nbase + c * CH, CH)],
                                dst.at[NCH // 2 + c])

        for c in range(NCH):
            pltpu.async_copy(rel_hbm.at[idx_r.at[c]],
                             rows_r.at[pl.ds(c * CH, CH)], sem)

        def fire_dim(d, carry):
            doff = d * R
            dbase = d * V
            for idx, cols in ((idx_h, cols_h), (idx_t, cols_t)):
                for c in range(NCH):
                    for j in range(CH // L):
                        iv = idx[c, pl.ds(j * L, L)] + dbase
                        pltpu.async_copy(
                            entT_hbm.at[iv],
                            cols.at[pl.ds(doff + c * CH + j * L, L)], sem)
            return carry

        lax.fori_loop(0, D, fire_dim, 0)

        pltpu.make_async_copy(entT_hbm.at[pl.ds(0, D * R)], cols_h, sem).wait()
        pltpu.make_async_copy(entT_hbm.at[pl.ds(0, D * R)], cols_t, sem).wait()
        pltpu.make_async_copy(rel_hbm.at[pl.ds(0, R)], rows_r, sem).wait()

        def half_scores(rbase):
            htr = h2 = t2 = jnp.zeros((L,), jnp.float32)
            for k in range(NCK):
                rT = _transpose16(
                    [rows_r[rbase + i, pl.ds(k * L, L)] for i in range(L)])
                for dd in range(L):
                    off = (k * L + dd) * R + rbase
                    hv = cols_h[pl.ds(off, L)]
                    tv = cols_t[pl.ds(off, L)]
                    htr = htr + hv * tv * rT[dd]
                    h2 = h2 + hv * hv
                    t2 = t2 + tv * tv
            return -(htr * _rsqrt_nr(jnp.maximum(h2, 1.0) *
                                     jnp.maximum(t2, 1.0)))

        def group(g, lacc):
            p_score = half_scores(g * L)
            n_score = half_scores(P + g * L)
            scores_v[pl.ds(g * L, L)] = p_score
            scores_v[pl.ds(P + g * L, L)] = n_score
            return lacc + jnp.maximum(p_score - n_score + margin, 0.0)

        lacc = lax.fori_loop(0, G, group, jnp.zeros((L,), jnp.float32))
        lrow_v[...] = jnp.full((L,), jnp.sum(lacc))
        pltpu.sync_copy(scores_v.at[pl.ds(0, P)], pos_out.at[pl.ds(pbase, P)])
        pltpu.sync_copy(scores_v.at[pl.ds(P, P)], neg_out.at[pl.ds(pbase, P)])
        pltpu.sync_copy(lrow_v, loss_out.at[wid])

    return distmult


def kernel(batch_h, batch_t, batch_r, batch_y, ent_emb, rel_emb):
    B = batch_h.shape[0]
    V, D = ent_emb.shape
    fn = _build(B, D, V, 1.0)
    ent_flat = ent_emb.T.reshape(-1)
    pos, neg, lpart = fn(batch_h.astype(jnp.int32), batch_t.astype(jnp.int32),
                         batch_r.astype(jnp.int32), ent_flat, rel_emb)
    loss = jnp.sum(lpart[:, 0])
    return (loss, pos, neg)
